# baseline (speedup 1.0000x reference)
import functools

import jax
import jax.numpy as jnp
from jax import lax
from jax.experimental import pallas as pl
from jax.experimental.pallas import tpu as pltpu
from jax.experimental.pallas import tpu_sc as plsc

N_FEATURES = 768
L1 = 1024
N_BUCKETS = 8
B = 16384
A = 32
TP = 776

NC = 2
NS = 16
NW = NC * NS
LANES = 16
CC = 256
TPW = TP // 4

NSPLIT = 2
HB = B // NSPLIT
RW = HB // NW

BB = 1024


def _sc_counts_body(stm_hbm, nstm_hbm, zeros_hbm, out_s, out_n, idx_v, cnt_v):
    wid = lax.axis_index("s") * NC + lax.axis_index("c")
    row0 = wid * RW
    pltpu.sync_copy(zeros_hbm, cnt_v)

    def scatter(sign):
        def grp(g, c2):
            v = idx_v[pl.ds(g * LANES, LANES)]
            row = jnp.zeros((LANES,), jnp.int32) + (g // 2)
            word = v >> 2
            add = sign << ((v & 3) << 3)
            plsc.addupdate_scatter(cnt_v, [row, word], add)
            return c2
        lax.fori_loop(0, CC * A // LANES, grp, 0)

    def side(idx_hbm, out_hbm):
        def chunk(k, carry):
            base = row0 + k * CC
            pltpu.sync_copy(idx_hbm.at[pl.ds(base * A, CC * A)], idx_v)
            scatter(jnp.int32(1))
            pltpu.sync_copy(cnt_v, out_hbm.at[pl.ds(base, CC)])
            scatter(jnp.int32(-1))
            return carry
        lax.fori_loop(0, RW // CC, chunk, 0)

    side(stm_hbm, out_s)
    side(nstm_hbm, out_n)


@functools.partial(
    pl.kernel,
    out_type=[jax.ShapeDtypeStruct((HB, TPW), jnp.int32),
              jax.ShapeDtypeStruct((HB, TPW), jnp.int32)],
    mesh=plsc.VectorSubcoreMesh(core_axis_name="c", subcore_axis_name="s"),
    scratch_types=[pltpu.VMEM((CC * A,), jnp.int32),
                   pltpu.VMEM((CC, TPW), jnp.int32)],
    compiler_params=pltpu.CompilerParams(needs_layout_passes=False),
)
def _sc_counts(*args):
    _sc_counts_body(*args)


def _dense_block(cs_ref, cn_ref, stm_ref, t0_ref, t1_ref, t2_ref, t3_ref,
                 bias_ref, w1_ref, w2_ref, b2_ref, out_ref):
    def half(c_ref, w_ref):
        c32 = c_ref[...]
        emb = bias_ref[...].astype(jnp.float32)
        for j, t_ref in enumerate((t0_ref, t1_ref, t2_ref, t3_ref)):
            cj = ((c32 >> (8 * j)) & 0xFF).astype(jnp.bfloat16)
            emb = emb + jnp.dot(cj, t_ref[...],
                                preferred_element_type=jnp.float32)
        h = jnp.clip(emb, 0.0, 1.0)
        h = h * h
        return jnp.dot(h.astype(jnp.bfloat16), w_ref[...],
                       preferred_element_type=jnp.float32)

    p = half(cs_ref, w1_ref) + half(cn_ref, w2_ref) + b2_ref[...]

    count = jnp.sum((stm_ref[...] != N_FEATURES).astype(jnp.int32), axis=1,
                    keepdims=True)
    bucket = ((count + 30) // 4) % N_BUCKETS
    sel = (bucket == lax.broadcasted_iota(jnp.int32, (BB, N_BUCKETS), 1))
    out_ref[...] = jnp.sum(p * sel.astype(jnp.float32), axis=1, keepdims=True)


def _dense_call(counts_s, counts_n, stm_h, tabs, bias2d, w1, w2, b2d):
    grid = (HB // BB,)
    return pl.pallas_call(
        _dense_block,
        grid=grid,
        in_specs=[
            pl.BlockSpec((BB, TPW), lambda i: (i, 0)),
            pl.BlockSpec((BB, TPW), lambda i: (i, 0)),
            pl.BlockSpec((BB, A), lambda i: (i, 0)),
            pl.BlockSpec((TPW, L1), lambda i: (0, 0)),
            pl.BlockSpec((TPW, L1), lambda i: (0, 0)),
            pl.BlockSpec((TPW, L1), lambda i: (0, 0)),
            pl.BlockSpec((TPW, L1), lambda i: (0, 0)),
            pl.BlockSpec((1, L1), lambda i: (0, 0)),
            pl.BlockSpec((L1, N_BUCKETS), lambda i: (0, 0)),
            pl.BlockSpec((L1, N_BUCKETS), lambda i: (0, 0)),
            pl.BlockSpec((1, N_BUCKETS), lambda i: (0, 0)),
        ],
        out_specs=pl.BlockSpec((BB, 1), lambda i: (i, 0)),
        out_shape=jax.ShapeDtypeStruct((HB, 1), jnp.float32),
    )(counts_s, counts_n, stm_h, *tabs, bias2d, w1, w2, b2d)


@jax.jit
def kernel(stm_indices, nstm_indices, table, input_bias, W, b):
    tab = table.at[N_FEATURES].set(0.0)
    tab = jnp.pad(tab, ((0, TP - (N_FEATURES + 1)), (0, 0)))
    tabs = [tab[j::4].astype(jnp.bfloat16) for j in range(4)]
    w1 = W[:, :L1].T.astype(jnp.bfloat16)
    w2 = W[:, L1:].T.astype(jnp.bfloat16)
    bias2d = input_bias[None, :]
    b2d = b[None, :]
    stm = stm_indices.astype(jnp.int32)
    nstm = nstm_indices.astype(jnp.int32)
    stm_flat = stm.reshape(B * A)
    nstm_flat = nstm.reshape(B * A)

    zeros = jnp.zeros((CC, TPW), jnp.int32)
    counts = [
        _sc_counts(stm_flat[h * HB * A:(h + 1) * HB * A],
                   nstm_flat[h * HB * A:(h + 1) * HB * A], zeros)
        for h in range(NSPLIT)
    ]
    outs = [
        _dense_call(counts[h][0], counts[h][1], stm[h * HB:(h + 1) * HB],
                    tabs, bias2d, w1, w2, b2d)
        for h in range(NSPLIT)
    ]
    return jnp.concatenate(outs, axis=0)


# speedup vs baseline: 1.0160x; 1.0160x over previous
import functools

import jax
import jax.numpy as jnp
from jax import lax
from jax.experimental import pallas as pl
from jax.experimental.pallas import tpu as pltpu
from jax.experimental.pallas import tpu_sc as plsc

N_FEATURES = 768
L1 = 1024
N_BUCKETS = 8
B = 16384
A = 32
TP = 776

NC = 2
NS = 16
NW = NC * NS
LANES = 16
CC = 256
TPW = TP // 4

NSPLIT = 2
HB = B // NSPLIT
RW = HB // NW

BB = 1024


assert RW == CC, "one counts chunk per worker per side"


def _sc_counts_body(stm_hbm, nstm_hbm, zeros_hbm, out_s, out_n, idx_v, cnt_v):
    wid = lax.axis_index("s") * NC + lax.axis_index("c")
    row0 = wid * RW
    pltpu.sync_copy(zeros_hbm, cnt_v)

    def scatter(sign):
        def grp(g, c2):
            v = idx_v[pl.ds(g * LANES, LANES)]
            row = jnp.zeros((LANES,), jnp.int32) + (g // 2)
            word = v >> 2
            add = sign << ((v & 3) << 3)
            plsc.addupdate_scatter(cnt_v, [row, word], add)
            return c2
        lax.fori_loop(0, CC * A // LANES, grp, 0)

    pltpu.sync_copy(stm_hbm.at[pl.ds(row0 * A, CC * A)], idx_v)
    scatter(jnp.int32(1))
    pltpu.sync_copy(cnt_v, out_s.at[pl.ds(row0, CC)])
    scatter(jnp.int32(-1))
    pltpu.sync_copy(nstm_hbm.at[pl.ds(row0 * A, CC * A)], idx_v)
    scatter(jnp.int32(1))
    pltpu.sync_copy(cnt_v, out_n.at[pl.ds(row0, CC)])


@functools.partial(
    pl.kernel,
    out_type=[jax.ShapeDtypeStruct((HB, TPW), jnp.int32),
              jax.ShapeDtypeStruct((HB, TPW), jnp.int32)],
    mesh=plsc.VectorSubcoreMesh(core_axis_name="c", subcore_axis_name="s"),
    scratch_types=[pltpu.VMEM((CC * A,), jnp.int32),
                   pltpu.VMEM((CC, TPW), jnp.int32)],
    compiler_params=pltpu.CompilerParams(needs_layout_passes=False),
)
def _sc_counts(*args):
    _sc_counts_body(*args)


def _dense_block(cs_ref, cn_ref, stm_ref, t0_ref, t1_ref, t2_ref, t3_ref,
                 bias_ref, w1_ref, w2_ref, b2_ref, out_ref):
    def half(c_ref, w_ref):
        c32 = c_ref[...]
        emb = bias_ref[...].astype(jnp.float32)
        for j, t_ref in enumerate((t0_ref, t1_ref, t2_ref, t3_ref)):
            cj = ((c32 >> (8 * j)) & 0xFF).astype(jnp.bfloat16)
            emb = emb + jnp.dot(cj, t_ref[...],
                                preferred_element_type=jnp.float32)
        h = jnp.clip(emb, 0.0, 1.0)
        h = h * h
        return jnp.dot(h.astype(jnp.bfloat16), w_ref[...],
                       preferred_element_type=jnp.float32)

    p = half(cs_ref, w1_ref) + half(cn_ref, w2_ref) + b2_ref[...]

    count = jnp.sum((stm_ref[...] != N_FEATURES).astype(jnp.int32), axis=1,
                    keepdims=True)
    bucket = ((count + 30) // 4) % N_BUCKETS
    sel = (bucket == lax.broadcasted_iota(jnp.int32, (BB, N_BUCKETS), 1))
    out_ref[...] = jnp.sum(p * sel.astype(jnp.float32), axis=1, keepdims=True)


def _dense_call(counts_s, counts_n, stm_h, tabs, bias2d, w1, w2, b2d):
    grid = (HB // BB,)
    return pl.pallas_call(
        _dense_block,
        grid=grid,
        in_specs=[
            pl.BlockSpec((BB, TPW), lambda i: (i, 0)),
            pl.BlockSpec((BB, TPW), lambda i: (i, 0)),
            pl.BlockSpec((BB, A), lambda i: (i, 0)),
            pl.BlockSpec((TPW, L1), lambda i: (0, 0)),
            pl.BlockSpec((TPW, L1), lambda i: (0, 0)),
            pl.BlockSpec((TPW, L1), lambda i: (0, 0)),
            pl.BlockSpec((TPW, L1), lambda i: (0, 0)),
            pl.BlockSpec((1, L1), lambda i: (0, 0)),
            pl.BlockSpec((L1, N_BUCKETS), lambda i: (0, 0)),
            pl.BlockSpec((L1, N_BUCKETS), lambda i: (0, 0)),
            pl.BlockSpec((1, N_BUCKETS), lambda i: (0, 0)),
        ],
        out_specs=pl.BlockSpec((BB, 1), lambda i: (i, 0)),
        out_shape=jax.ShapeDtypeStruct((HB, 1), jnp.float32),
    )(counts_s, counts_n, stm_h, *tabs, bias2d, w1, w2, b2d)


@jax.jit
def kernel(stm_indices, nstm_indices, table, input_bias, W, b):
    tab = table.at[N_FEATURES].set(0.0)
    tab = jnp.pad(tab, ((0, TP - (N_FEATURES + 1)), (0, 0)))
    tabs = [tab[j::4].astype(jnp.bfloat16) for j in range(4)]
    w1 = W[:, :L1].T.astype(jnp.bfloat16)
    w2 = W[:, L1:].T.astype(jnp.bfloat16)
    bias2d = input_bias[None, :]
    b2d = b[None, :]
    stm = stm_indices.astype(jnp.int32)
    nstm = nstm_indices.astype(jnp.int32)
    stm_flat = stm.reshape(B * A)
    nstm_flat = nstm.reshape(B * A)

    zeros = jnp.zeros((CC, TPW), jnp.int32)
    counts = [
        _sc_counts(stm_flat[h * HB * A:(h + 1) * HB * A],
                   nstm_flat[h * HB * A:(h + 1) * HB * A], zeros)
        for h in range(NSPLIT)
    ]
    outs = [
        _dense_call(counts[h][0], counts[h][1], stm[h * HB:(h + 1) * HB],
                    tabs, bias2d, w1, w2, b2d)
        for h in range(NSPLIT)
    ]
    return jnp.concatenate(outs, axis=0)
